# Initial kernel scaffold; baseline (speedup 1.0000x reference)
import jax
import jax.numpy as jnp
from jax.experimental import pallas as pl


def kernel(users, items, graph_edge_index, graph_edge_weight, user_emb, item_emb, W1, b1, W2, b2, Wo, bo):
    raise NotImplementedError("write your pallas kernel here")


# speedup vs baseline: 10.6779x; 10.6779x over previous
import functools

import jax
import jax.numpy as jnp
from jax import lax
from jax.experimental import pallas as pl
from jax.experimental.pallas import tpu as pltpu
from jax.experimental.pallas import tpu_sc as plsc

NU = 50000
NI = 50000
NN = NU + NI
NP = 102400
EE = 1600000
DD = 32
HH = 16
NLAYERS = 3
BB = 4096

NSUB = 16
EPT = 102400
EP = EPT * NSUB
CHUNK = 1024
RPC = CHUNK // 128
NCHUNK = EPT // CHUNK
ROWS_PER_SUB = NP // NSUB
ZROWS = 160
SEL = 2 * BB
SPS = SEL // NSUB


def _sc_body(e0, srcx, dstx, wv, selx, flat, selout,
             acc, src_v, dst_v, w_v, rows_v, zed_v, sid_v, aid_v,
             buf0, buf1, sem):
    c = lax.axis_index("c")
    s = lax.axis_index("s")
    iota = lax.iota(jnp.int32, 16)
    cN = c * NP

    def zinit(i, carry):
        zed_v[i, :] = jnp.zeros((16,), jnp.float32)
        return carry
    lax.fori_loop(0, ZROWS, zinit, 0)

    for l in range(NLAYERS):
        for z in range(ROWS_PER_SUB // ZROWS):
            pltpu.sync_copy(zed_v, acc.at[pl.ds(s * ROWS_PER_SUB + z * ZROWS, ZROWS)])
        plsc.subcore_barrier()

        tb = e0 if l == 0 else flat
        toff = cN if l == 0 else (l - 1) * 2 * NP + cN
        toff_vec = jnp.full((16,), 0, jnp.int32) + toff

        def chunk_body(k, carry):
            rbase = s * (EPT // 128) + k * RPC
            pltpu.sync_copy(srcx.at[pl.ds(rbase, RPC)], src_v)
            pltpu.sync_copy(dstx.at[pl.ds(rbase, RPC)], dst_v)
            wbase = pl.multiple_of(rbase * 128, CHUNK)
            pltpu.sync_copy(wv.at[pl.ds(wbase, CHUNK)], w_v)

            def adj(r, carry2):
                for j in range(8):
                    src_v[r, pl.ds(j * 16, 16)] = (
                        src_v[r, pl.ds(j * 16, 16)] + toff_vec)
                return carry2
            lax.fori_loop(0, RPC, adj, 0)

            descs = []
            for r in range(RPC):
                descs.append(pltpu.async_copy(
                    tb.at[src_v.at[r]], rows_v.at[pl.ds(r * 128, 128)], sem))
            for d in descs:
                d.wait()

            def scale(g, carry2):
                gbase = pl.multiple_of(g * 16, 16)
                wvec = w_v[pl.ds(gbase, 16)]
                for j in range(16):
                    e = gbase + j
                    wj = lax.gather(
                        wvec, jnp.full((16, 1), j, jnp.int32),
                        dimension_numbers=lax.GatherDimensionNumbers(
                            offset_dims=(), collapsed_slice_dims=(0,),
                            start_index_map=(0,)),
                        slice_sizes=(1,),
                        mode=lax.GatherScatterMode.PROMISE_IN_BOUNDS)
                    rows_v[e, :] = rows_v[e, :] * wj
                return carry2
            lax.fori_loop(0, CHUNK // 16, scale, 0)

            descs2 = []
            for r in range(RPC):
                descs2.append(pltpu.async_copy(
                    rows_v.at[pl.ds(r * 128, 128)], acc.at[dst_v.at[r]],
                    sem, add=True))
            for d in descs2:
                d.wait()
            return carry
        lax.fori_loop(0, NCHUNK, chunk_body, 0)
        plsc.subcore_barrier()

        pltpu.sync_copy(
            acc.at[pl.ds(s * ROWS_PER_SUB, ROWS_PER_SUB)],
            flat.at[pl.ds(l * 2 * NP + cN + s * ROWS_PER_SUB, ROWS_PER_SUB)])

    plsc.subcore_barrier()

    pltpu.sync_copy(selx.at[pl.ds(s * (SPS // 128), SPS // 128)], sid_v)
    for r in range(SPS // 128):
        row = r
        for j in range(8):
            aid_v[pl.ds(j * 16, 16)] = (
                sid_v[row, pl.ds(j * 16, 16)] + (jnp.full((16,), 0, jnp.int32) + cN))
        pltpu.async_copy(e0.at[aid_v], buf0, sem).wait()
        for l in range(1, NLAYERS + 1):
            off2 = (l - 1) * 2 * NP
            for j in range(8):
                aid_v[pl.ds(j * 16, 16)] = (
                    sid_v[row, pl.ds(j * 16, 16)]
                    + (jnp.full((16,), 0, jnp.int32) + (off2 + cN)))
            pltpu.async_copy(flat.at[aid_v], buf1, sem).wait()
            last = (l == NLAYERS)

            def addb(i, carry, _last=last):
                if _last:
                    buf0[i, :] = (buf0[i, :] + buf1[i, :]) * 0.25
                else:
                    buf0[i, :] = buf0[i, :] + buf1[i, :]
                return carry
            lax.fori_loop(0, 128, addb, 0)
        pltpu.sync_copy(buf0, selout.at[pl.ds(c * SEL + s * SPS + r * 128, 128)])


def _propagate(e0, srcx, dstx, wv, selx):
    mesh = plsc.VectorSubcoreMesh(core_axis_name="c", subcore_axis_name="s")
    f = pl.kernel(
        _sc_body,
        out_type=(
            jax.ShapeDtypeStruct((NLAYERS * 2 * NP, HH), jnp.float32),
            jax.ShapeDtypeStruct((2 * SEL, HH), jnp.float32),
        ),
        mesh=mesh,
        scratch_types=[
            pltpu.VMEM_SHARED((NP, HH), jnp.float32),
            pltpu.VMEM((RPC, 128), jnp.int32),
            pltpu.VMEM((RPC, 128), jnp.int32),
            pltpu.VMEM((CHUNK,), jnp.float32),
            pltpu.VMEM((CHUNK, HH), jnp.float32),
            pltpu.VMEM((ZROWS, HH), jnp.float32),
            pltpu.VMEM((SPS // 128, 128), jnp.int32),
            pltpu.VMEM((128,), jnp.int32),
            pltpu.VMEM((128, HH), jnp.float32),
            pltpu.VMEM((128, HH), jnp.float32),
            pltpu.SemaphoreType.DMA,
        ],
        compiler_params=pltpu.CompilerParams(use_tc_tiling_on_sc=False),
    )
    return f(e0, srcx, dstx, wv, selx)


def _mlp_body(x_ref, w1_ref, b1_ref, w2_ref, b2_ref, wo_ref, bo_ref, o_ref):
    x = x_ref[:]
    h = jnp.maximum(
        jnp.dot(x, w1_ref[:], preferred_element_type=jnp.float32) + b1_ref[:], 0.0)
    h = jnp.maximum(
        jnp.dot(h, w2_ref[:], preferred_element_type=jnp.float32) + b2_ref[:], 0.0)
    z = jnp.dot(h, wo_ref[:], preferred_element_type=jnp.float32) + bo_ref[:]
    o_ref[:] = jax.nn.sigmoid(z)


def _mlp(x, W1, b1, W2, b2, Wo, bo):
    return pl.pallas_call(
        _mlp_body,
        out_shape=jax.ShapeDtypeStruct((BB, 1), jnp.float32),
    )(x, W1, b1, W2, b2, Wo, bo)


def kernel(users, items, graph_edge_index, graph_edge_weight,
           user_emb, item_emb, W1, b1, W2, b2, Wo, bo):
    i32 = jnp.int32
    src = graph_edge_index[0].astype(i32)
    dst = graph_edge_index[1].astype(i32)
    w = graph_edge_weight.astype(jnp.float32)
    pad = EP - EE
    src = jnp.concatenate([src, jnp.zeros((pad,), i32)])
    dst = jnp.concatenate([dst, jnp.zeros((pad,), i32)])
    w = jnp.concatenate([w, jnp.zeros((pad,), jnp.float32)])
    srcx = src.reshape(EP // 128, 128)
    dstx = dst.reshape(EP // 128, 128)

    all_emb = jnp.concatenate([user_emb, item_emb], 0)
    zrows = jnp.zeros((NP - NN, HH), jnp.float32)
    e0 = jnp.concatenate([all_emb[:, :HH], zrows,
                          all_emb[:, HH:], zrows], 0)
    selx = jnp.concatenate([users.astype(i32), items.astype(i32) + NU]
                           ).reshape(SEL // 128, 128)

    _, selsum = _propagate(e0, srcx, dstx, w, selx)
    rows32 = jnp.concatenate([selsum[:SEL], selsum[SEL:]], 1)
    vec = jnp.concatenate([rows32[:BB], rows32[BB:]], 1)
    return _mlp(vec, W1, b1.reshape(1, -1), W2, b2.reshape(1, -1),
                Wo, bo.reshape(1, -1))
